# Initial kernel scaffold; baseline (speedup 1.0000x reference)
import jax
import jax.numpy as jnp
from jax.experimental import pallas as pl


def kernel(ids, table):
    raise NotImplementedError("write your pallas kernel here")


# speedup vs baseline: 5.1416x; 5.1416x over previous
import functools

import jax
import jax.numpy as jnp
from jax import lax
from jax.experimental import pallas as pl
from jax.experimental.pallas import tpu as pltpu
from jax.experimental.pallas import tpu_sc as plsc

_NC, _NS = 2, 16
_NW = _NC * _NS
_CHUNK = 512


def kernel(ids, table):
    B, T = ids.shape
    V, D = table.shape
    N = B * T
    assert N % (_NW * _CHUNK) == 0
    b_per_w = N // _NW
    n_chunks = b_per_w // _CHUNK
    flat_ids = ids.reshape(N)
    table128 = jnp.pad(table, ((0, 0), (0, 128 - D)))

    mesh = plsc.VectorSubcoreMesh(core_axis_name="c", subcore_axis_name="s")

    @functools.partial(
        pl.kernel,
        mesh=mesh,
        out_type=jax.ShapeDtypeStruct((N, 128), table.dtype),
        scratch_types=[
            pltpu.VMEM((_CHUNK,), jnp.int32),
            pltpu.VMEM((_CHUNK, 128), jnp.float32),
            pltpu.SemaphoreType.DMA,
        ],
    )
    def k(table_hbm, idx_hbm, out_hbm, idx_v, rows_v, sem):
        wid = lax.axis_index("s") * _NC + lax.axis_index("c")
        base = wid * b_per_w

        @pl.loop(0, n_chunks)
        def _(i):
            off = base + i * _CHUNK
            pltpu.sync_copy(idx_hbm.at[pl.ds(off, _CHUNK)], idx_v)
            pltpu.async_copy(table_hbm.at[idx_v], rows_v, sem).wait()
            pltpu.sync_copy(rows_v, out_hbm.at[pl.ds(off, _CHUNK)])

    out = k(table128, flat_ids)
    return out[:, :D].reshape(B, T, D)


# speedup vs baseline: 5.5923x; 1.0876x over previous
import functools

import jax
import jax.numpy as jnp
from jax import lax
from jax.experimental import pallas as pl
from jax.experimental.pallas import tpu as pltpu
from jax.experimental.pallas import tpu_sc as plsc

_NC, _NS = 2, 16
_NW = _NC * _NS
_CHUNK = 320


def kernel(ids, table):
    B, T = ids.shape
    V, D = table.shape
    N = B * T
    assert N % (_NW * 2 * _CHUNK) == 0
    b_per_w = N // _NW
    n_chunks = b_per_w // _CHUNK
    n2 = n_chunks // 2
    flat_ids = ids.reshape(N)
    table128 = jnp.pad(table, ((0, 0), (0, 128 - D)))

    mesh = plsc.VectorSubcoreMesh(core_axis_name="c", subcore_axis_name="s")

    @functools.partial(
        pl.kernel,
        mesh=mesh,
        out_type=jax.ShapeDtypeStruct((N, 128), table.dtype),
        scratch_types=[
            pltpu.VMEM((b_per_w,), jnp.int32),
            pltpu.VMEM((_CHUNK, 128), jnp.float32),
            pltpu.VMEM((_CHUNK, 128), jnp.float32),
            pltpu.SemaphoreType.DMA,
            pltpu.SemaphoreType.DMA,
            pltpu.SemaphoreType.DMA,
            pltpu.SemaphoreType.DMA,
        ],
    )
    def k(table_hbm, idx_hbm, out_hbm, idx_all, rows0, rows1, g0, g1, s0, s1):
        wid = lax.axis_index("s") * _NC + lax.axis_index("c")
        base = wid * b_per_w
        pltpu.sync_copy(idx_hbm.at[pl.ds(base, b_per_w)], idx_all)

        def gather_desc(i, buf, sem):
            return pltpu.make_async_copy(
                table_hbm.at[idx_all.at[pl.ds(i * _CHUNK, _CHUNK)]], buf, sem)

        def store_desc(i, buf, sem):
            return pltpu.make_async_copy(
                buf, out_hbm.at[pl.ds(base + i * _CHUNK, _CHUNK)], sem)

        gather_desc(0, rows0, g0).start()

        @pl.loop(0, n2)
        def _(j):
            i = 2 * j
            @pl.when(j > 0)
            def _():
                store_desc(i - 1, rows1, s1).wait()

            gather_desc(i + 1, rows1, g1).start()
            gather_desc(i, rows0, g0).wait()
            store_desc(i, rows0, s0).start()

            @pl.when(j < n2 - 1)
            def _():
                store_desc(i, rows0, s0).wait()
                gather_desc(i + 2, rows0, g0).start()

            gather_desc(i + 1, rows1, g1).wait()
            store_desc(i + 1, rows1, s1).start()

        store_desc(n_chunks - 2, rows0, s0).wait()
        store_desc(n_chunks - 1, rows1, s1).wait()

    out = k(table128, flat_ids)
    return out[:, :D].reshape(B, T, D)
